# Initial kernel scaffold; baseline (speedup 1.0000x reference)
import functools

import jax
import jax.numpy as jnp
from jax import lax
from jax.experimental import pallas as pl
from jax.experimental.pallas import tpu as pltpu
from jax.experimental.pallas import tpu_sc as plsc

N = 10000
E = 320000
D = 128

NC = 2
NS = 16
NW = NC * NS
K = 128
NCH = -(-E // (NW * K))
EPW = NCH * K
E_PAD = NW * EPW
N_PAD = 10240
RPS = N_PAD // NS
SLABS = RPS // K


def _seg_body(with_deg, *refs):
    if with_deg:
        (h_hbm, src_hbm, dst_hbm, out_hbm, deg_hbm,
         src_v, dst_v, rows_v, zbuf_v, acc_sh, sem,
         ones_v, z1_v, dacc_sh) = refs
    else:
        (h_hbm, src_hbm, dst_hbm, out_hbm,
         src_v, dst_v, rows_v, zbuf_v, acc_sh, sem) = refs

    c = lax.axis_index("c")
    s = lax.axis_index("s")
    wid = s * NC + c

    zv = jnp.zeros((16,), jnp.float32)

    def zrow(r, carry):
        for g in range(D // 16):
            zbuf_v[r, pl.ds(g * 16, 16)] = zv
        return carry

    lax.fori_loop(0, K, zrow, 0)
    for i in range(SLABS):
        pltpu.sync_copy(zbuf_v, acc_sh.at[pl.ds(s * RPS + i * K, K)])

    if with_deg:
        ov = jnp.ones((16,), jnp.float32)
        for g in range(K // 16):
            ones_v[pl.ds(g * 16, 16)] = ov
        for g in range(RPS // 16):
            z1_v[pl.ds(g * 16, 16)] = zv
        pltpu.sync_copy(z1_v, dacc_sh.at[pl.ds(s * RPS, RPS)])

    plsc.subcore_barrier()

    pltpu.sync_copy(src_hbm.at[wid], src_v)
    pltpu.sync_copy(dst_hbm.at[wid], dst_v)

    def step(j, carry):
        pltpu.async_copy(h_hbm.at[src_v.at[j]], rows_v, sem).wait()
        pltpu.sync_copy(rows_v, acc_sh.at[dst_v.at[j]], add=True)
        if with_deg:
            pltpu.sync_copy(ones_v, dacc_sh.at[dst_v.at[j]], add=True)
        return carry

    lax.fori_loop(0, NCH, step, 0)

    plsc.subcore_barrier()

    for i in range(SLABS):
        pltpu.sync_copy(acc_sh.at[pl.ds(s * RPS + i * K, K)], zbuf_v)
        pltpu.sync_copy(zbuf_v, out_hbm.at[pl.ds(c * N_PAD + s * RPS + i * K, K)])
    if with_deg:
        pltpu.sync_copy(dacc_sh.at[pl.ds(s * RPS, RPS)], z1_v)
        pltpu.sync_copy(z1_v, deg_hbm.at[pl.ds(c * N_PAD + s * RPS, RPS)])


def _make_seg(with_deg):
    out_type = [jax.ShapeDtypeStruct((NC * N_PAD, D), jnp.float32)]
    scratch = [
        pltpu.VMEM((NCH, K), jnp.int32),
        pltpu.VMEM((NCH, K), jnp.int32),
        pltpu.VMEM((K, D), jnp.float32),
        pltpu.VMEM((K, D), jnp.float32),
        pltpu.VMEM_SHARED((N_PAD, D), jnp.float32),
        pltpu.SemaphoreType.DMA,
    ]
    if with_deg:
        out_type.append(jax.ShapeDtypeStruct((NC * N_PAD,), jnp.float32))
        scratch += [
            pltpu.VMEM((K,), jnp.float32),
            pltpu.VMEM((RPS,), jnp.float32),
            pltpu.VMEM_SHARED((N_PAD,), jnp.float32),
        ]
    return pl.kernel(
        functools.partial(_seg_body, with_deg),
        out_type=tuple(out_type),
        mesh=plsc.VectorSubcoreMesh(core_axis_name="c", subcore_axis_name="s"),
        scratch_types=tuple(scratch),
    )


_seg0 = _make_seg(True)
_seg1 = _make_seg(False)

_BK = 2000
_GRID = N // _BK


def _row_spec():
    return pl.BlockSpec((_BK, D), lambda i: (i, 0))


def _full_spec():
    return pl.BlockSpec((D, D), lambda i: (0, 0))


def _one_spec():
    return pl.BlockSpec((_BK, 1), lambda i: (i, 0))


def _bias_spec():
    return pl.BlockSpec((1, D), lambda i: (0, 0))


def _log1p_body(x_ref, o_ref):
    o_ref[...] = jnp.log(x_ref[...] + 1.0)


def _tc_log1p(x):
    return pl.pallas_call(
        _log1p_body,
        grid=(_GRID,),
        in_specs=[_row_spec()],
        out_specs=_row_spec(),
        out_shape=jax.ShapeDtypeStruct((N, D), jnp.float32),
    )(x)


def _layer0_body(h_ref, p0_ref, p1_ref, d0_ref, d1_ref, ws_ref, wn_ref, b_ref,
                 o_ref):
    deg = jnp.maximum(d0_ref[...] + d1_ref[...], 1.0)
    agg = (p0_ref[...] + p1_ref[...]) / deg
    z = (jnp.dot(h_ref[...], ws_ref[...], preferred_element_type=jnp.float32)
         + jnp.dot(agg, wn_ref[...], preferred_element_type=jnp.float32)
         + b_ref[...])
    z = jnp.maximum(z, 0.0)
    nrm = jnp.sqrt(jnp.sum(z * z, axis=1, keepdims=True))
    o_ref[...] = z / jnp.maximum(nrm, 1e-12)


def _tc_layer0(h, p0, p1, d0, d1, ws, wn, b):
    return pl.pallas_call(
        _layer0_body,
        grid=(_GRID,),
        in_specs=[_row_spec(), _row_spec(), _row_spec(), _one_spec(),
                  _one_spec(), _full_spec(), _full_spec(), _bias_spec()],
        out_specs=_row_spec(),
        out_shape=jax.ShapeDtypeStruct((N, D), jnp.float32),
    )(h, p0, p1, d0, d1, ws, wn, b)


def _tail_body(h_ref, p0_ref, p1_ref, d0_ref, d1_ref, ws_ref, wn_ref, b_ref,
               wfc_ref, bfc_ref, g_ref, beta_ref, w21_ref, b21_ref, w22_ref,
               b22_ref, loc_ref, scale_ref):
    deg = jnp.maximum(d0_ref[...] + d1_ref[...], 1.0)
    agg = (p0_ref[...] + p1_ref[...]) / deg
    h2 = (jnp.dot(h_ref[...], ws_ref[...], preferred_element_type=jnp.float32)
          + jnp.dot(agg, wn_ref[...], preferred_element_type=jnp.float32)
          + b_ref[...])
    t = jnp.dot(h2, wfc_ref[...], preferred_element_type=jnp.float32) + bfc_ref[...]
    t = t * (1.0 / jnp.sqrt(1.0 + 1e-5)) * g_ref[...] + beta_ref[...]
    t = jnp.maximum(t, 0.0)
    t = t + jnp.log1p(jnp.exp(-t))
    loc_ref[...] = (jnp.dot(t, w21_ref[...], preferred_element_type=jnp.float32)
                    + b21_ref[...])
    scale_ref[...] = jnp.exp(
        jnp.dot(t, w22_ref[...], preferred_element_type=jnp.float32)
        + b22_ref[...])


def _tc_tail(h, p0, p1, d0, d1, ws, wn, b, wfc, bfc, g, beta, w21, b21, w22,
             b22):
    return pl.pallas_call(
        _tail_body,
        grid=(_GRID,),
        in_specs=[_row_spec(), _row_spec(), _row_spec(), _one_spec(),
                  _one_spec(), _full_spec(), _full_spec(), _bias_spec(),
                  _full_spec(), _bias_spec(), _bias_spec(), _bias_spec(),
                  _full_spec(), _bias_spec(), _full_spec(), _bias_spec()],
        out_specs=[_row_spec(), _row_spec()],
        out_shape=[jax.ShapeDtypeStruct((N, D), jnp.float32),
                   jax.ShapeDtypeStruct((N, D), jnp.float32)],
    )(h, p0, p1, d0, d1, ws, wn, b, wfc, bfc, g, beta, w21, b21, w22, b22)


def kernel(x, edge_index, W_self0, W_neigh0, b0, W_self1, W_neigh1, b1,
           W_fc, b_fc, bn_gamma, bn_beta, W21, b21, W22, b22):
    src = edge_index[0]
    dst = edge_index[1]
    pad = E_PAD - E
    ar = jnp.arange(pad, dtype=jnp.int32)
    psrc = (ar * 131) % N
    pdst = N + ar % (N_PAD - N)
    src_r = jnp.concatenate([src, psrc]).reshape(NW, NCH, K)
    dst_r = jnp.concatenate([dst, pdst]).reshape(NW, NCH, K)

    b0r = b0.reshape(1, D)
    b1r = b1.reshape(1, D)
    bfcr = b_fc.reshape(1, D)
    gr = bn_gamma.reshape(1, D)
    betar = bn_beta.reshape(1, D)
    b21r = b21.reshape(1, D)
    b22r = b22.reshape(1, D)

    h0 = _tc_log1p(x)
    part0, deg = _seg0(h0, src_r, dst_r)
    d0 = deg[:N].reshape(N, 1)
    d1 = deg[N_PAD:N_PAD + N].reshape(N, 1)
    h1 = _tc_layer0(h0, part0[:N], part0[N_PAD:N_PAD + N], d0, d1,
                    W_self0, W_neigh0, b0r)
    (part1,) = _seg1(h1, src_r, dst_r)
    loc, scale = _tc_tail(h1, part1[:N], part1[N_PAD:N_PAD + N], d0, d1,
                          W_self1, W_neigh1, b1r, W_fc, bfcr, gr, betar,
                          W21, b21r, W22, b22r)
    return (loc, scale)


# speedup vs baseline: 7.8883x; 7.8883x over previous
import functools

import jax
import jax.numpy as jnp
from jax import lax
from jax.experimental import pallas as pl
from jax.experimental.pallas import tpu as pltpu
from jax.experimental.pallas import tpu_sc as plsc

N = 10000
E = 320000
D = 128

NC = 2
NS = 16
NW = NC * NS
K = 128
NCH = -(-E // (NW * K))
EPW = NCH * K
E_PAD = NW * EPW
N_PAD = 10240
RPS = N_PAD // NS
SLABS = RPS // K


def _seg_body(with_deg, *refs):
    if with_deg:
        (h_hbm, src_hbm, dst_hbm, out_hbm, deg_hbm,
         src_v, dst_v, rows_v, acc_sh, sem,
         ones_v, z1_v, dacc_sh) = refs
    else:
        (h_hbm, src_hbm, dst_hbm, out_hbm,
         src_v, dst_v, rows_v, acc_sh, sem) = refs

    c = lax.axis_index("c")
    s = lax.axis_index("s")
    wid = s * NC + c

    zv = jnp.zeros((16,), jnp.float32)

    def zrow(r, carry):
        for g in range(D // 16):
            rows_v[r, pl.ds(g * 16, 16)] = zv
        return carry

    lax.fori_loop(0, K, zrow, 0)
    for i in range(SLABS):
        pltpu.sync_copy(rows_v, acc_sh.at[pl.ds(s * RPS + i * K, K)])

    if with_deg:
        ov = jnp.ones((16,), jnp.float32)
        for g in range(K // 16):
            ones_v[pl.ds(g * 16, 16)] = ov
        for g in range(RPS // 16):
            z1_v[pl.ds(g * 16, 16)] = zv
        pltpu.sync_copy(z1_v, dacc_sh.at[pl.ds(s * RPS, RPS)])

    plsc.subcore_barrier()

    pltpu.sync_copy(src_hbm.at[wid], src_v)
    pltpu.sync_copy(dst_hbm.at[wid], dst_v)

    def step(j, carry):
        pltpu.async_copy(h_hbm.at[src_v.at[j]], rows_v, sem).wait()
        pltpu.sync_copy(rows_v, acc_sh.at[dst_v.at[j]], add=True)
        if with_deg:
            pltpu.sync_copy(ones_v, dacc_sh.at[dst_v.at[j]], add=True)
        return carry

    lax.fori_loop(0, NCH, step, 0)

    plsc.subcore_barrier()

    for i in range(SLABS):
        pltpu.sync_copy(acc_sh.at[pl.ds(s * RPS + i * K, K)], rows_v)
        pltpu.sync_copy(rows_v, out_hbm.at[pl.ds(c * N_PAD + s * RPS + i * K, K)])
    if with_deg:
        pltpu.sync_copy(dacc_sh.at[pl.ds(s * RPS, RPS)], z1_v)
        pltpu.sync_copy(z1_v, deg_hbm.at[pl.ds(c * N_PAD + s * RPS, RPS)])


def _make_seg(with_deg):
    out_type = [jax.ShapeDtypeStruct((NC * N_PAD, D), jnp.float32)]
    scratch = [
        pltpu.VMEM((NCH, K), jnp.int32),
        pltpu.VMEM((NCH, K), jnp.int32),
        pltpu.VMEM((K, D), jnp.float32),
        pltpu.VMEM_SHARED((N_PAD, D), jnp.float32),
        pltpu.SemaphoreType.DMA,
    ]
    if with_deg:
        out_type.append(jax.ShapeDtypeStruct((NC * N_PAD,), jnp.float32))
        scratch += [
            pltpu.VMEM((K,), jnp.float32),
            pltpu.VMEM((RPS,), jnp.float32),
            pltpu.VMEM_SHARED((N_PAD,), jnp.float32),
        ]
    return pl.kernel(
        functools.partial(_seg_body, with_deg),
        out_type=tuple(out_type),
        mesh=plsc.VectorSubcoreMesh(core_axis_name="c", subcore_axis_name="s"),
        scratch_types=tuple(scratch),
    )


_seg0 = _make_seg(True)
_seg1 = _make_seg(False)

_BK = 2000
_GRID = N // _BK


def _row_spec():
    return pl.BlockSpec((_BK, D), lambda i: (i, 0))


def _full_spec():
    return pl.BlockSpec((D, D), lambda i: (0, 0))


def _one_spec():
    return pl.BlockSpec((_BK, 1), lambda i: (i, 0))


def _bias_spec():
    return pl.BlockSpec((1, D), lambda i: (0, 0))


def _log1p_body(x_ref, o_ref):
    o_ref[...] = jnp.log(x_ref[...] + 1.0)


def _tc_log1p(x):
    return pl.pallas_call(
        _log1p_body,
        grid=(_GRID,),
        in_specs=[_row_spec()],
        out_specs=_row_spec(),
        out_shape=jax.ShapeDtypeStruct((N, D), jnp.float32),
    )(x)


def _layer0_body(h_ref, p0_ref, p1_ref, d0_ref, d1_ref, ws_ref, wn_ref, b_ref,
                 o_ref):
    deg = jnp.maximum(d0_ref[...] + d1_ref[...], 1.0)
    agg = (p0_ref[...] + p1_ref[...]) / deg
    z = (jnp.dot(h_ref[...], ws_ref[...], preferred_element_type=jnp.float32)
         + jnp.dot(agg, wn_ref[...], preferred_element_type=jnp.float32)
         + b_ref[...])
    z = jnp.maximum(z, 0.0)
    nrm = jnp.sqrt(jnp.sum(z * z, axis=1, keepdims=True))
    o_ref[...] = z / jnp.maximum(nrm, 1e-12)


def _tc_layer0(h, p0, p1, d0, d1, ws, wn, b):
    return pl.pallas_call(
        _layer0_body,
        grid=(_GRID,),
        in_specs=[_row_spec(), _row_spec(), _row_spec(), _one_spec(),
                  _one_spec(), _full_spec(), _full_spec(), _bias_spec()],
        out_specs=_row_spec(),
        out_shape=jax.ShapeDtypeStruct((N, D), jnp.float32),
    )(h, p0, p1, d0, d1, ws, wn, b)


def _tail_body(h_ref, p0_ref, p1_ref, d0_ref, d1_ref, ws_ref, wn_ref, b_ref,
               wfc_ref, bfc_ref, g_ref, beta_ref, w21_ref, b21_ref, w22_ref,
               b22_ref, loc_ref, scale_ref):
    deg = jnp.maximum(d0_ref[...] + d1_ref[...], 1.0)
    agg = (p0_ref[...] + p1_ref[...]) / deg
    h2 = (jnp.dot(h_ref[...], ws_ref[...], preferred_element_type=jnp.float32)
          + jnp.dot(agg, wn_ref[...], preferred_element_type=jnp.float32)
          + b_ref[...])
    t = jnp.dot(h2, wfc_ref[...], preferred_element_type=jnp.float32) + bfc_ref[...]
    t = t * (1.0 / jnp.sqrt(1.0 + 1e-5)) * g_ref[...] + beta_ref[...]
    t = jnp.maximum(t, 0.0)
    t = t + jnp.log1p(jnp.exp(-t))
    loc_ref[...] = (jnp.dot(t, w21_ref[...], preferred_element_type=jnp.float32)
                    + b21_ref[...])
    scale_ref[...] = jnp.exp(
        jnp.dot(t, w22_ref[...], preferred_element_type=jnp.float32)
        + b22_ref[...])


def _tc_tail(h, p0, p1, d0, d1, ws, wn, b, wfc, bfc, g, beta, w21, b21, w22,
             b22):
    return pl.pallas_call(
        _tail_body,
        grid=(_GRID,),
        in_specs=[_row_spec(), _row_spec(), _row_spec(), _one_spec(),
                  _one_spec(), _full_spec(), _full_spec(), _bias_spec(),
                  _full_spec(), _bias_spec(), _bias_spec(), _bias_spec(),
                  _full_spec(), _bias_spec(), _full_spec(), _bias_spec()],
        out_specs=[_row_spec(), _row_spec()],
        out_shape=[jax.ShapeDtypeStruct((N, D), jnp.float32),
                   jax.ShapeDtypeStruct((N, D), jnp.float32)],
    )(h, p0, p1, d0, d1, ws, wn, b, wfc, bfc, g, beta, w21, b21, w22, b22)


def kernel(x, edge_index, W_self0, W_neigh0, b0, W_self1, W_neigh1, b1,
           W_fc, b_fc, bn_gamma, bn_beta, W21, b21, W22, b22):
    src = edge_index[0]
    dst = edge_index[1]
    pad = E_PAD - E
    ar = jnp.arange(pad, dtype=jnp.int32)
    psrc = (ar * 131) % N
    pdst = N + ar % (N_PAD - N)
    src_r = jnp.concatenate([src, psrc]).reshape(NW, NCH, K)
    dst_r = jnp.concatenate([dst, pdst]).reshape(NW, NCH, K)

    b0r = b0.reshape(1, D)
    b1r = b1.reshape(1, D)
    bfcr = b_fc.reshape(1, D)
    gr = bn_gamma.reshape(1, D)
    betar = bn_beta.reshape(1, D)
    b21r = b21.reshape(1, D)
    b22r = b22.reshape(1, D)

    h0 = _tc_log1p(x)
    part0, deg = _seg0(h0, src_r, dst_r)
    d0 = deg[:N].reshape(N, 1)
    d1 = deg[N_PAD:N_PAD + N].reshape(N, 1)
    h1 = _tc_layer0(h0, part0[:N], part0[N_PAD:N_PAD + N], d0, d1,
                    W_self0, W_neigh0, b0r)
    (part1,) = _seg1(h1, src_r, dst_r)
    loc, scale = _tc_tail(h1, part1[:N], part1[N_PAD:N_PAD + N], d0, d1,
                          W_self1, W_neigh1, b1r, W_fc, bfcr, gr, betar,
                          W21, b21r, W22, b22r)
    return (loc, scale)


# speedup vs baseline: 11.5067x; 1.4587x over previous
import functools

import jax
import jax.numpy as jnp
from jax import lax
from jax.experimental import pallas as pl
from jax.experimental.pallas import tpu as pltpu
from jax.experimental.pallas import tpu_sc as plsc

N = 10000
E = 320000
D = 128

NC = 2
NS = 16
NW = NC * NS
K = 128
G = 8
NGRP = 10
NSG = NGRP // 2
NCH = NGRP * G
EPW = NCH * K
E_PAD = NW * EPW
N_PAD = 10240
RPS = N_PAD // NS
SLABS = RPS // K


def _seg_body(with_deg, *refs):
    if with_deg:
        (h_hbm, src_hbm, dst_hbm, out_hbm, deg_hbm,
         sbuf0, dbuf0, sbuf1, dbuf1, rbuf0, rbuf1,
         acc_sh, sem_i0, sem_i1, sem_g0, sem_g1,
         ones_v, z1_v, dacc_sh) = refs
    else:
        (h_hbm, src_hbm, dst_hbm, out_hbm,
         sbuf0, dbuf0, sbuf1, dbuf1, rbuf0, rbuf1,
         acc_sh, sem_i0, sem_i1, sem_g0, sem_g1) = refs

    c = lax.axis_index("c")
    s = lax.axis_index("s")
    wid = s * NC + c

    sbufs = (sbuf0, sbuf1)
    dbufs = (dbuf0, dbuf1)
    rbufs = (rbuf0, rbuf1)
    sem_is = (sem_i0, sem_i1)
    sem_gs = (sem_g0, sem_g1)

    zv = jnp.zeros((16,), jnp.float32)

    def zrow(r, carry):
        for g in range(D // 16):
            rbuf0[r, pl.ds(g * 16, 16)] = zv
        return carry

    lax.fori_loop(0, K, zrow, 0)
    for i in range(SLABS):
        pltpu.sync_copy(rbuf0, acc_sh.at[pl.ds(s * RPS + i * K, K)])

    if with_deg:
        ov = jnp.ones((16,), jnp.float32)
        for g in range(K // 16):
            ones_v[pl.ds(g * 16, 16)] = ov
        for g in range(RPS // 16):
            z1_v[pl.ds(g * 16, 16)] = zv
        pltpu.sync_copy(z1_v, dacc_sh.at[pl.ds(s * RPS, RPS)])

    plsc.subcore_barrier()

    def idx_group(g):
        return pl.ds(wid * NCH + g * G, G)

    def wait_idx(h):
        pltpu.make_async_copy(src_hbm.at[pl.ds(0, G)], sbufs[h], sem_is[h]).wait()
        pltpu.make_async_copy(dst_hbm.at[pl.ds(0, G)], dbufs[h], sem_is[h]).wait()

    def wait_gather(p):
        pltpu.make_async_copy(h_hbm.at[pl.ds(0, K)], rbufs[p], sem_gs[p]).wait()

    pltpu.sync_copy(src_hbm.at[idx_group(0)], sbuf0)
    pltpu.sync_copy(dst_hbm.at[idx_group(0)], dbuf0)
    pltpu.async_copy(src_hbm.at[idx_group(1)], sbuf1, sem_i1)
    pltpu.async_copy(dst_hbm.at[idx_group(1)], dbuf1, sem_i1)
    pltpu.async_copy(h_hbm.at[sbuf0.at[0]], rbuf0, sem_g0)

    def supergroup(sg, carry):
        for half in range(2):
            g = 2 * sg + half
            sb, db = sbufs[half], dbufs[half]
            nsb = sbufs[1 - half]
            for r in range(G):
                p = r % 2
                if r < G - 1:
                    pltpu.async_copy(h_hbm.at[sb.at[r + 1]], rbufs[1 - p],
                                     sem_gs[1 - p])
                else:
                    @pl.when(g + 1 < NGRP)
                    def _():
                        wait_idx(1 - half)
                        pltpu.async_copy(h_hbm.at[nsb.at[0]], rbufs[1 - p],
                                         sem_gs[1 - p])
                wait_gather(p)
                pltpu.sync_copy(rbufs[p], acc_sh.at[db.at[r]], add=True)
                if with_deg:
                    pltpu.sync_copy(ones_v, dacc_sh.at[db.at[r]], add=True)
                if r == G - 1:
                    @pl.when(g + 2 < NGRP)
                    def _():
                        pltpu.async_copy(src_hbm.at[idx_group(g + 2)], sb,
                                         sem_is[half])
                        pltpu.async_copy(dst_hbm.at[idx_group(g + 2)], db,
                                         sem_is[half])
        return carry

    lax.fori_loop(0, NSG, supergroup, 0)

    plsc.subcore_barrier()

    for i in range(SLABS):
        pltpu.sync_copy(acc_sh.at[pl.ds(s * RPS + i * K, K)], rbuf0)
        pltpu.sync_copy(rbuf0, out_hbm.at[pl.ds(c * N_PAD + s * RPS + i * K, K)])
    if with_deg:
        pltpu.sync_copy(dacc_sh.at[pl.ds(s * RPS, RPS)], z1_v)
        pltpu.sync_copy(z1_v, deg_hbm.at[pl.ds(c * N_PAD + s * RPS, RPS)])


def _make_seg(with_deg):
    out_type = [jax.ShapeDtypeStruct((NC * N_PAD, D), jnp.float32)]
    scratch = [
        pltpu.VMEM((G, K), jnp.int32),
        pltpu.VMEM((G, K), jnp.int32),
        pltpu.VMEM((G, K), jnp.int32),
        pltpu.VMEM((G, K), jnp.int32),
        pltpu.VMEM((K, D), jnp.float32),
        pltpu.VMEM((K, D), jnp.float32),
        pltpu.VMEM_SHARED((N_PAD, D), jnp.float32),
        pltpu.SemaphoreType.DMA,
        pltpu.SemaphoreType.DMA,
        pltpu.SemaphoreType.DMA,
        pltpu.SemaphoreType.DMA,
    ]
    if with_deg:
        out_type.append(jax.ShapeDtypeStruct((NC * N_PAD,), jnp.float32))
        scratch += [
            pltpu.VMEM((K,), jnp.float32),
            pltpu.VMEM((RPS,), jnp.float32),
            pltpu.VMEM_SHARED((N_PAD,), jnp.float32),
        ]
    return pl.kernel(
        functools.partial(_seg_body, with_deg),
        out_type=tuple(out_type),
        mesh=plsc.VectorSubcoreMesh(core_axis_name="c", subcore_axis_name="s"),
        scratch_types=tuple(scratch),
    )


_seg0 = _make_seg(True)
_seg1 = _make_seg(False)

_BK = 2000
_GRID = N // _BK


def _row_spec():
    return pl.BlockSpec((_BK, D), lambda i: (i, 0))


def _full_spec():
    return pl.BlockSpec((D, D), lambda i: (0, 0))


def _one_spec():
    return pl.BlockSpec((_BK, 1), lambda i: (i, 0))


def _bias_spec():
    return pl.BlockSpec((1, D), lambda i: (0, 0))


def _log1p_body(x_ref, o_ref):
    o_ref[...] = jnp.log(x_ref[...] + 1.0)


def _tc_log1p(x):
    return pl.pallas_call(
        _log1p_body,
        grid=(_GRID,),
        in_specs=[_row_spec()],
        out_specs=_row_spec(),
        out_shape=jax.ShapeDtypeStruct((N, D), jnp.float32),
    )(x)


def _layer0_body(h_ref, p0_ref, p1_ref, d0_ref, d1_ref, ws_ref, wn_ref, b_ref,
                 o_ref):
    deg = jnp.maximum(d0_ref[...] + d1_ref[...], 1.0)
    agg = (p0_ref[...] + p1_ref[...]) / deg
    z = (jnp.dot(h_ref[...], ws_ref[...], preferred_element_type=jnp.float32)
         + jnp.dot(agg, wn_ref[...], preferred_element_type=jnp.float32)
         + b_ref[...])
    z = jnp.maximum(z, 0.0)
    nrm = jnp.sqrt(jnp.sum(z * z, axis=1, keepdims=True))
    o_ref[...] = z / jnp.maximum(nrm, 1e-12)


def _tc_layer0(h, p0, p1, d0, d1, ws, wn, b):
    return pl.pallas_call(
        _layer0_body,
        grid=(_GRID,),
        in_specs=[_row_spec(), _row_spec(), _row_spec(), _one_spec(),
                  _one_spec(), _full_spec(), _full_spec(), _bias_spec()],
        out_specs=_row_spec(),
        out_shape=jax.ShapeDtypeStruct((N, D), jnp.float32),
    )(h, p0, p1, d0, d1, ws, wn, b)


def _tail_body(h_ref, p0_ref, p1_ref, d0_ref, d1_ref, ws_ref, wn_ref, b_ref,
               wfc_ref, bfc_ref, g_ref, beta_ref, w21_ref, b21_ref, w22_ref,
               b22_ref, loc_ref, scale_ref):
    deg = jnp.maximum(d0_ref[...] + d1_ref[...], 1.0)
    agg = (p0_ref[...] + p1_ref[...]) / deg
    h2 = (jnp.dot(h_ref[...], ws_ref[...], preferred_element_type=jnp.float32)
          + jnp.dot(agg, wn_ref[...], preferred_element_type=jnp.float32)
          + b_ref[...])
    t = jnp.dot(h2, wfc_ref[...], preferred_element_type=jnp.float32) + bfc_ref[...]
    t = t * (1.0 / jnp.sqrt(1.0 + 1e-5)) * g_ref[...] + beta_ref[...]
    t = jnp.maximum(t, 0.0)
    t = t + jnp.log1p(jnp.exp(-t))
    loc_ref[...] = (jnp.dot(t, w21_ref[...], preferred_element_type=jnp.float32)
                    + b21_ref[...])
    scale_ref[...] = jnp.exp(
        jnp.dot(t, w22_ref[...], preferred_element_type=jnp.float32)
        + b22_ref[...])


def _tc_tail(h, p0, p1, d0, d1, ws, wn, b, wfc, bfc, g, beta, w21, b21, w22,
             b22):
    return pl.pallas_call(
        _tail_body,
        grid=(_GRID,),
        in_specs=[_row_spec(), _row_spec(), _row_spec(), _one_spec(),
                  _one_spec(), _full_spec(), _full_spec(), _bias_spec(),
                  _full_spec(), _bias_spec(), _bias_spec(), _bias_spec(),
                  _full_spec(), _bias_spec(), _full_spec(), _bias_spec()],
        out_specs=[_row_spec(), _row_spec()],
        out_shape=[jax.ShapeDtypeStruct((N, D), jnp.float32),
                   jax.ShapeDtypeStruct((N, D), jnp.float32)],
    )(h, p0, p1, d0, d1, ws, wn, b, wfc, bfc, g, beta, w21, b21, w22, b22)


def kernel(x, edge_index, W_self0, W_neigh0, b0, W_self1, W_neigh1, b1,
           W_fc, b_fc, bn_gamma, bn_beta, W21, b21, W22, b22):
    src = edge_index[0]
    dst = edge_index[1]
    pad = E_PAD - E
    ar = jnp.arange(pad, dtype=jnp.int32)
    psrc = (ar * 131) % N
    pdst = N + ar % (N_PAD - N)
    src_r = jnp.concatenate([src, psrc]).reshape(NW * NCH, K)
    dst_r = jnp.concatenate([dst, pdst]).reshape(NW * NCH, K)

    b0r = b0.reshape(1, D)
    b1r = b1.reshape(1, D)
    bfcr = b_fc.reshape(1, D)
    gr = bn_gamma.reshape(1, D)
    betar = bn_beta.reshape(1, D)
    b21r = b21.reshape(1, D)
    b22r = b22.reshape(1, D)

    h0 = _tc_log1p(x)
    part0, deg = _seg0(h0, src_r, dst_r)
    d0 = deg[:N].reshape(N, 1)
    d1 = deg[N_PAD:N_PAD + N].reshape(N, 1)
    h1 = _tc_layer0(h0, part0[:N], part0[N_PAD:N_PAD + N], d0, d1,
                    W_self0, W_neigh0, b0r)
    (part1,) = _seg1(h1, src_r, dst_r)
    loc, scale = _tc_tail(h1, part1[:N], part1[N_PAD:N_PAD + N], d0, d1,
                          W_self1, W_neigh1, b1r, W_fc, bfcr, gr, betar,
                          W21, b21r, W22, b22r)
    return (loc, scale)


# speedup vs baseline: 12.0274x; 1.0452x over previous
import functools

import jax
import jax.numpy as jnp
from jax import lax
from jax.experimental import pallas as pl
from jax.experimental.pallas import tpu as pltpu
from jax.experimental.pallas import tpu_sc as plsc

N = 10000
E = 320000
D = 128

NC = 2
NS = 16
NW = NC * NS
K = 128
G = 8
NGRP = 10
NSG = NGRP // 2
NCH = NGRP * G
EPW = NCH * K
E_PAD = NW * EPW
N_PAD = 10240
RPS = N_PAD // NS
SLABS = RPS // K


def _seg_body(with_deg, *refs):
    if with_deg:
        (h_hbm, src_hbm, dst_hbm, out_hbm, deg_hbm,
         sbuf0, dbuf0, sbuf1, dbuf1, rbuf0, rbuf1,
         acc_sh, sem_i0, sem_i1, sem_g0, sem_g1, sem_s0, sem_s1,
         ones_v, z1_v, dacc_sh) = refs
    else:
        (h_hbm, src_hbm, dst_hbm, out_hbm,
         sbuf0, dbuf0, sbuf1, dbuf1, rbuf0, rbuf1,
         acc_sh, sem_i0, sem_i1, sem_g0, sem_g1, sem_s0, sem_s1) = refs

    c = lax.axis_index("c")
    s = lax.axis_index("s")
    wid = s * NC + c

    sbufs = (sbuf0, sbuf1)
    dbufs = (dbuf0, dbuf1)
    rbufs = (rbuf0, rbuf1)
    sem_is = (sem_i0, sem_i1)
    sem_gs = (sem_g0, sem_g1)
    sem_ss = (sem_s0, sem_s1)

    zv = jnp.zeros((16,), jnp.float32)

    def zrow(r, carry):
        for g in range(D // 16):
            rbuf0[r, pl.ds(g * 16, 16)] = zv
        return carry

    lax.fori_loop(0, K, zrow, 0)
    for i in range(SLABS):
        pltpu.sync_copy(rbuf0, acc_sh.at[pl.ds(s * RPS + i * K, K)])

    if with_deg:
        ov = jnp.ones((16,), jnp.float32)
        for g in range(K // 16):
            ones_v[pl.ds(g * 16, 16)] = ov
        for g in range(RPS // 16):
            z1_v[pl.ds(g * 16, 16)] = zv
        pltpu.sync_copy(z1_v, dacc_sh.at[pl.ds(s * RPS, RPS)])

    plsc.subcore_barrier()

    def idx_group(g):
        return pl.ds(wid * NCH + g * G, G)

    def wait_idx(h):
        pltpu.make_async_copy(src_hbm.at[pl.ds(0, G)], sbufs[h], sem_is[h]).wait()
        pltpu.make_async_copy(dst_hbm.at[pl.ds(0, G)], dbufs[h], sem_is[h]).wait()

    def wait_gather(p):
        pltpu.make_async_copy(h_hbm.at[pl.ds(0, K)], rbufs[p], sem_gs[p]).wait()

    def wait_scatter(p):
        pltpu.make_async_copy(rbufs[p], acc_sh.at[pl.ds(0, K)], sem_ss[p]).wait()

    pltpu.sync_copy(src_hbm.at[idx_group(0)], sbuf0)
    pltpu.sync_copy(dst_hbm.at[idx_group(0)], dbuf0)
    pltpu.async_copy(src_hbm.at[idx_group(1)], sbuf1, sem_i1)
    pltpu.async_copy(dst_hbm.at[idx_group(1)], dbuf1, sem_i1)
    pltpu.async_copy(h_hbm.at[sbuf0.at[0]], rbuf0, sem_g0)

    def supergroup(sg, carry):
        for half in range(2):
            g = 2 * sg + half
            sb, db = sbufs[half], dbufs[half]
            nsb = sbufs[1 - half]
            for r in range(G):
                p = r % 2
                if r == 0:
                    @pl.when(g > 0)
                    def _():
                        wait_scatter(1 - p)
                    pltpu.async_copy(h_hbm.at[sb.at[1]], rbufs[1 - p],
                                     sem_gs[1 - p])
                elif r < G - 1:
                    wait_scatter(1 - p)
                    pltpu.async_copy(h_hbm.at[sb.at[r + 1]], rbufs[1 - p],
                                     sem_gs[1 - p])
                else:
                    @pl.when(g + 1 < NGRP)
                    def _():
                        wait_scatter(1 - p)
                        wait_idx(1 - half)
                        pltpu.async_copy(h_hbm.at[nsb.at[0]], rbufs[1 - p],
                                         sem_gs[1 - p])
                wait_gather(p)
                pltpu.async_copy(rbufs[p], acc_sh.at[db.at[r]], sem_ss[p],
                                 add=True)
                if with_deg:
                    pltpu.sync_copy(ones_v, dacc_sh.at[db.at[r]], add=True)
                if r == G - 1:
                    @pl.when(g + 2 < NGRP)
                    def _():
                        pltpu.async_copy(src_hbm.at[idx_group(g + 2)], sb,
                                         sem_is[half])
                        pltpu.async_copy(dst_hbm.at[idx_group(g + 2)], db,
                                         sem_is[half])
        return carry

    lax.fori_loop(0, NSG, supergroup, 0)

    wait_scatter(0)
    wait_scatter(1)

    plsc.subcore_barrier()

    for i in range(SLABS):
        pltpu.sync_copy(acc_sh.at[pl.ds(s * RPS + i * K, K)], rbuf0)
        pltpu.sync_copy(rbuf0, out_hbm.at[c, pl.ds(s * RPS + i * K, K)])
    if with_deg:
        pltpu.sync_copy(dacc_sh.at[pl.ds(s * RPS, RPS)], z1_v)
        pltpu.sync_copy(z1_v, deg_hbm.at[pl.ds(c * N_PAD + s * RPS, RPS)])


def _make_seg(with_deg):
    out_type = [jax.ShapeDtypeStruct((NC, N_PAD, D), jnp.float32)]
    scratch = [
        pltpu.VMEM((G, K), jnp.int32),
        pltpu.VMEM((G, K), jnp.int32),
        pltpu.VMEM((G, K), jnp.int32),
        pltpu.VMEM((G, K), jnp.int32),
        pltpu.VMEM((K, D), jnp.float32),
        pltpu.VMEM((K, D), jnp.float32),
        pltpu.VMEM_SHARED((N_PAD, D), jnp.float32),
        pltpu.SemaphoreType.DMA,
        pltpu.SemaphoreType.DMA,
        pltpu.SemaphoreType.DMA,
        pltpu.SemaphoreType.DMA,
        pltpu.SemaphoreType.DMA,
        pltpu.SemaphoreType.DMA,
    ]
    if with_deg:
        out_type.append(jax.ShapeDtypeStruct((NC * N_PAD,), jnp.float32))
        scratch += [
            pltpu.VMEM((K,), jnp.float32),
            pltpu.VMEM((RPS,), jnp.float32),
            pltpu.VMEM_SHARED((N_PAD,), jnp.float32),
        ]
    return pl.kernel(
        functools.partial(_seg_body, with_deg),
        out_type=tuple(out_type),
        mesh=plsc.VectorSubcoreMesh(core_axis_name="c", subcore_axis_name="s"),
        scratch_types=tuple(scratch),
    )


_seg0 = _make_seg(True)
_seg1 = _make_seg(False)

_BK = 2000
_GRID = N // _BK


def _row_spec():
    return pl.BlockSpec((_BK, D), lambda i: (i, 0))


def _part_spec(core):
    return pl.BlockSpec((1, _BK, D), lambda i, core=core: (core, i, 0))


def _deg_spec(core):
    return pl.BlockSpec((1, _BK, 1), lambda i, core=core: (core, i, 0))


def _full_spec():
    return pl.BlockSpec((D, D), lambda i: (0, 0))


def _bias_spec():
    return pl.BlockSpec((1, D), lambda i: (0, 0))


def _log1p_body(x_ref, o_ref):
    o_ref[...] = jnp.log(x_ref[...] + 1.0)


def _tc_log1p(x):
    return pl.pallas_call(
        _log1p_body,
        grid=(_GRID,),
        in_specs=[_row_spec()],
        out_specs=_row_spec(),
        out_shape=jax.ShapeDtypeStruct((N, D), jnp.float32),
    )(x)


def _agg(p0_ref, p1_ref, d0_ref, d1_ref):
    deg = jnp.maximum(d0_ref[0] + d1_ref[0], 1.0)
    return (p0_ref[0] + p1_ref[0]) / deg


def _layer0_body(h_ref, p0_ref, p1_ref, d0_ref, d1_ref, ws_ref, wn_ref, b_ref,
                 o_ref):
    agg = _agg(p0_ref, p1_ref, d0_ref, d1_ref)
    z = (jnp.dot(h_ref[...], ws_ref[...], preferred_element_type=jnp.float32)
         + jnp.dot(agg, wn_ref[...], preferred_element_type=jnp.float32)
         + b_ref[...])
    z = jnp.maximum(z, 0.0)
    nrm = jnp.sqrt(jnp.sum(z * z, axis=1, keepdims=True))
    o_ref[...] = z / jnp.maximum(nrm, 1e-12)


def _tc_layer0(h, part, deg3, ws, wn, b):
    return pl.pallas_call(
        _layer0_body,
        grid=(_GRID,),
        in_specs=[_row_spec(), _part_spec(0), _part_spec(1), _deg_spec(0),
                  _deg_spec(1), _full_spec(), _full_spec(), _bias_spec()],
        out_specs=_row_spec(),
        out_shape=jax.ShapeDtypeStruct((N, D), jnp.float32),
    )(h, part, part, deg3, deg3, ws, wn, b)


def _tail_body(h_ref, p0_ref, p1_ref, d0_ref, d1_ref, ws_ref, wn_ref, b_ref,
               wfc_ref, bfc_ref, g_ref, beta_ref, w21_ref, b21_ref, w22_ref,
               b22_ref, loc_ref, scale_ref):
    agg = _agg(p0_ref, p1_ref, d0_ref, d1_ref)
    h2 = (jnp.dot(h_ref[...], ws_ref[...], preferred_element_type=jnp.float32)
          + jnp.dot(agg, wn_ref[...], preferred_element_type=jnp.float32)
          + b_ref[...])
    t = jnp.dot(h2, wfc_ref[...], preferred_element_type=jnp.float32) + bfc_ref[...]
    t = t * (1.0 / jnp.sqrt(1.0 + 1e-5)) * g_ref[...] + beta_ref[...]
    t = jnp.maximum(t, 0.0)
    t = t + jnp.log1p(jnp.exp(-t))
    loc_ref[...] = (jnp.dot(t, w21_ref[...], preferred_element_type=jnp.float32)
                    + b21_ref[...])
    scale_ref[...] = jnp.exp(
        jnp.dot(t, w22_ref[...], preferred_element_type=jnp.float32)
        + b22_ref[...])


def _tc_tail(h, part, deg3, ws, wn, b, wfc, bfc, g, beta, w21, b21, w22, b22):
    return pl.pallas_call(
        _tail_body,
        grid=(_GRID,),
        in_specs=[_row_spec(), _part_spec(0), _part_spec(1), _deg_spec(0),
                  _deg_spec(1), _full_spec(), _full_spec(), _bias_spec(),
                  _full_spec(), _bias_spec(), _bias_spec(), _bias_spec(),
                  _full_spec(), _bias_spec(), _full_spec(), _bias_spec()],
        out_specs=[_row_spec(), _row_spec()],
        out_shape=[jax.ShapeDtypeStruct((N, D), jnp.float32),
                   jax.ShapeDtypeStruct((N, D), jnp.float32)],
    )(h, part, part, deg3, deg3, ws, wn, b, wfc, bfc, g, beta, w21, b21, w22,
      b22)


def kernel(x, edge_index, W_self0, W_neigh0, b0, W_self1, W_neigh1, b1,
           W_fc, b_fc, bn_gamma, bn_beta, W21, b21, W22, b22):
    src = edge_index[0]
    dst = edge_index[1]
    pad = E_PAD - E
    ar = jnp.arange(pad, dtype=jnp.int32)
    psrc = (ar * 131) % N
    pdst = N + ar % (N_PAD - N)
    src_r = jnp.concatenate([src, psrc]).reshape(NW * NCH, K)
    dst_r = jnp.concatenate([dst, pdst]).reshape(NW * NCH, K)

    b0r = b0.reshape(1, D)
    b1r = b1.reshape(1, D)
    bfcr = b_fc.reshape(1, D)
    gr = bn_gamma.reshape(1, D)
    betar = bn_beta.reshape(1, D)
    b21r = b21.reshape(1, D)
    b22r = b22.reshape(1, D)

    h0 = _tc_log1p(x)
    part0, deg = _seg0(h0, src_r, dst_r)
    deg3 = deg.reshape(NC, N_PAD, 1)
    h1 = _tc_layer0(h0, part0, deg3, W_self0, W_neigh0, b0r)
    (part1,) = _seg1(h1, src_r, dst_r)
    loc, scale = _tc_tail(h1, part1, deg3, W_self1, W_neigh1, b1r, W_fc, bfcr,
                          gr, betar, W21, b21r, W22, b22r)
    return (loc, scale)
